# Initial kernel scaffold; baseline (speedup 1.0000x reference)
import jax
import jax.numpy as jnp
from jax.experimental import pallas as pl


def kernel(lr_data, hr_data, lr_doy, hr_doy, target_doy):
    raise NotImplementedError("write your pallas kernel here")


# speedup vs baseline: 2.1243x; 2.1243x over previous
import numpy as np
import jax
import jax.numpy as jnp
from jax.experimental import pallas as pl
from jax.experimental.pallas import tpu as pltpu

_UP = 4


def _resize_mat(n_in: int, n_out: int) -> np.ndarray:
    scale = n_out / n_in
    sample = (np.arange(n_out) + 0.5) / scale - 0.5
    dist = np.abs(sample[None, :] - np.arange(n_in)[:, None])
    w = np.maximum(0.0, 1.0 - dist)
    w = w / w.sum(axis=0, keepdims=True)
    return w.astype(np.float32)


def _fusion_body(slr0, slr1, shr0, shr1, lr_doy_s, hr_doy_s, tgt_s,
                 lr0, lr1, hr0, hr1, a_ref, at_ref, out_lr, out_hr):
    b = pl.program_id(0)
    t = pl.program_id(1)
    tval = tgt_s[t].astype(jnp.float32)

    def lerp_weight(doy_s, i0, i1):
        d0 = doy_s[b, i0].astype(jnp.float32)
        d1 = doy_s[b, i1].astype(jnp.float32)
        denom = jnp.where(d1 - d0 == 0.0, 1.0, d1 - d0)
        return jnp.clip((tval - d0) / denom, 0.0, 1.0)

    w_hr = lerp_weight(hr_doy_s, shr0[b, t], shr1[b, t])
    out_hr[0, 0] = hr0[0, 0] * (1.0 - w_hr) + hr1[0, 0] * w_hr

    w_lr = lerp_weight(lr_doy_s, slr0[b, t], slr1[b, t])
    x = lr0[0, 0] * (1.0 - w_lr) + lr1[0, 0] * w_lr
    a = a_ref[...]
    at = at_ref[...]
    for c in range(x.shape[0]):
        y1 = jnp.dot(x[c], a, preferred_element_type=jnp.float32)
        out_lr[0, 0, c] = jnp.dot(at, y1, preferred_element_type=jnp.float32)


def kernel(lr_data, hr_data, lr_doy, hr_doy, target_doy):
    B, Tlr, C, Hl, Wl = lr_data.shape
    _, Thr, _, Hh, Wh = hr_data.shape
    Tt = target_doy.shape[0]
    Hu, Wu = Hl * _UP, Wl * _UP

    def bounds(doy):
        idx = jax.vmap(
            lambda d: jnp.searchsorted(d, target_doy, side='left'))(doy)
        i1 = jnp.clip(idx, 1, doy.shape[1] - 1).astype(jnp.int32)
        return i1 - 1, i1

    lr_i0, lr_i1 = bounds(lr_doy)
    hr_i0, hr_i1 = bounds(hr_doy)

    a_np = _resize_mat(Hl, Hu)
    a = jnp.asarray(a_np)
    at = jnp.asarray(np.ascontiguousarray(a_np.T))

    grid_spec = pltpu.PrefetchScalarGridSpec(
        num_scalar_prefetch=7,
        grid=(B, Tt),
        in_specs=[
            pl.BlockSpec((1, 1, C, Hl, Wl),
                         lambda b, t, *s: (b, s[0][b, t], 0, 0, 0)),
            pl.BlockSpec((1, 1, C, Hl, Wl),
                         lambda b, t, *s: (b, s[1][b, t], 0, 0, 0)),
            pl.BlockSpec((1, 1, C, Hh, Wh),
                         lambda b, t, *s: (b, s[2][b, t], 0, 0, 0)),
            pl.BlockSpec((1, 1, C, Hh, Wh),
                         lambda b, t, *s: (b, s[3][b, t], 0, 0, 0)),
            pl.BlockSpec((Hl, Hu), lambda b, t, *s: (0, 0)),
            pl.BlockSpec((Hu, Hl), lambda b, t, *s: (0, 0)),
        ],
        out_specs=[
            pl.BlockSpec((1, 1, C, Hu, Wu), lambda b, t, *s: (b, t, 0, 0, 0)),
            pl.BlockSpec((1, 1, C, Hh, Wh), lambda b, t, *s: (b, t, 0, 0, 0)),
        ],
    )

    out_lr, out_hr = pl.pallas_call(
        _fusion_body,
        grid_spec=grid_spec,
        out_shape=[
            jax.ShapeDtypeStruct((B, Tt, C, Hu, Wu), jnp.float32),
            jax.ShapeDtypeStruct((B, Tt, C, Hh, Wh), jnp.float32),
        ],
        compiler_params=pltpu.CompilerParams(
            dimension_semantics=("arbitrary", "arbitrary")),
    )(lr_i0, lr_i1, hr_i0, hr_i1, lr_doy, hr_doy, target_doy,
      lr_data, lr_data, hr_data, hr_data, a, at)
    return out_lr, out_hr
